# baseline (speedup 1.0000x reference)
import functools

import jax
import jax.numpy as jnp
from jax import lax
from jax.experimental import pallas as pl
from jax.experimental.pallas import tpu as pltpu
from jax.experimental.pallas import tpu_sc as plsc

LNEPS = 1e-5
_NW = 32
_CH = 80
_P = 8


def _sc_gather(table, idx):
    n = idx.shape[0]
    d = table.shape[1]
    per_w = n // _NW
    n_ch = per_w // _CH
    assert per_w % _CH == 0 and n_ch % 2 == 0 and n % _NW == 0
    idx3 = idx.reshape(_NW, n_ch, _CH)
    mesh = plsc.VectorSubcoreMesh(core_axis_name="c", subcore_axis_name="s")

    @functools.partial(
        pl.kernel,
        mesh=mesh,
        out_type=jax.ShapeDtypeStruct((n, d), jnp.float32),
        scratch_types=[
            pltpu.VMEM((n_ch, _CH), jnp.int32),
            pltpu.VMEM((_CH, d), jnp.float32),
            pltpu.VMEM((_CH, d), jnp.float32),
            pltpu.SemaphoreType.DMA,
            pltpu.SemaphoreType.DMA,
            pltpu.SemaphoreType.DMA,
            pltpu.SemaphoreType.DMA,
        ],
    )
    def gather_kernel(table_hbm, idx_hbm, out_hbm, idx_v, buf0, buf1,
                      sg0, sg1, ss0, ss1):
        wid = lax.axis_index("s") * 2 + lax.axis_index("c")
        base = wid * per_w
        pltpu.sync_copy(idx_hbm.at[wid], idx_v)

        def body(g, _):
            j0 = 2 * g
            j1 = j0 + 1
            g0 = pltpu.make_async_copy(table_hbm.at[idx_v.at[j0]], buf0, sg0)
            g1 = pltpu.make_async_copy(table_hbm.at[idx_v.at[j1]], buf1, sg1)
            g0.start()
            g1.start()
            g0.wait()
            s0 = pltpu.make_async_copy(
                buf0, out_hbm.at[pl.ds(base + j0 * _CH, _CH)], ss0)
            s0.start()
            g1.wait()
            s1 = pltpu.make_async_copy(
                buf1, out_hbm.at[pl.ds(base + j1 * _CH, _CH)], ss1)
            s1.start()
            s0.wait()
            s1.wait()
            return 0

        lax.fori_loop(0, n_ch // 2, body, 0)

    return gather_kernel(table, idx3)


def _tc_fuse_chunk(xg, tids3, pos_table, type_table, gamma2, beta2, mask3,
                   b, s, d, bp, off, carry):
    g = 32
    ntypes = type_table.shape[0]
    ob = off // g

    def body(*refs):
        (x_ref, tid_ref, pos_ref, typ_ref, g_ref, b_ref, m_ref) = refs[:7]
        o_ref, mo_ref = refs[-2:]
        x = x_ref[...].reshape(g, s, d)
        emb = x + pos_ref[...].reshape(1, s, d)
        tid = tid_ref[...].astype(jnp.int32)
        for k in range(ntypes):
            row = typ_ref[k:k + 1, :].reshape(1, 1, d)
            emb = emb + jnp.where(tid == k, 1.0, 0.0) * row
        mean = jnp.mean(emb, axis=-1, keepdims=True)
        c = emb - mean
        var = jnp.mean(c * c, axis=-1, keepdims=True)
        y = c * lax.rsqrt(var + LNEPS)
        y = y * g_ref[...].reshape(1, 1, d) + b_ref[...].reshape(1, 1, d)
        o_ref[...] = y
        mo_ref[...] = ((1.0 - m_ref[...]) * -10000.0).reshape(g, 1, 1, s)

    in_specs = [
        pl.BlockSpec((g * s, d), lambda i: (i, 0)),
        pl.BlockSpec((g, s, 1), lambda i: (i + ob, 0, 0)),
        pl.BlockSpec((s, d), lambda i: (0, 0)),
        pl.BlockSpec((ntypes, d), lambda i: (0, 0)),
        pl.BlockSpec((1, d), lambda i: (0, 0)),
        pl.BlockSpec((1, d), lambda i: (0, 0)),
        pl.BlockSpec((g, 1, s), lambda i: (i + ob, 0, 0)),
    ]
    args = [xg, tids3, pos_table, type_table, gamma2, beta2, mask3]
    aliases = {}
    if carry is not None:
        in_specs += [pl.BlockSpec(memory_space=pl.ANY),
                     pl.BlockSpec(memory_space=pl.ANY)]
        args += [carry[0], carry[1]]
        aliases = {7: 0, 8: 1}

    return pl.pallas_call(
        body,
        grid=(bp // g,),
        in_specs=in_specs,
        out_specs=[
            pl.BlockSpec((g, s, d), lambda i: (i + ob, 0, 0)),
            pl.BlockSpec((g, 1, 1, s), lambda i: (i + ob, 0, 0, 0)),
        ],
        out_shape=[
            jax.ShapeDtypeStruct((b, s, d), jnp.float32),
            jax.ShapeDtypeStruct((b, 1, 1, s), jnp.float32),
        ],
        input_output_aliases=aliases,
    )(*args)


def kernel(input_ids, domain_type_ids, position_ids, attention_mask,
           token_table, pos_table, type_table, gamma, beta):
    b, s = input_ids.shape
    d = token_table.shape[1]
    bp = b // _P
    ids = input_ids.astype(jnp.int32)
    tids3 = domain_type_ids.astype(jnp.int8).reshape(b, s, 1)
    mask3 = attention_mask.astype(jnp.float32).reshape(b, 1, s)
    gamma2 = gamma.reshape(1, d)
    beta2 = beta.reshape(1, d)

    carry = None
    for p in range(_P):
        ids_p = lax.slice_in_dim(ids, p * bp, (p + 1) * bp, axis=0)
        g_p = _sc_gather(token_table, ids_p.reshape(-1))
        carry = _tc_fuse_chunk(g_p, tids3, pos_table, type_table,
                               gamma2, beta2, mask3, b, s, d,
                               bp, p * bp, carry)
    return (carry[0], carry[1])


# speedup vs baseline: 1.1186x; 1.1186x over previous
import functools

import jax
import jax.numpy as jnp
from jax import lax
from jax.experimental import pallas as pl
from jax.experimental.pallas import tpu as pltpu
from jax.experimental.pallas import tpu_sc as plsc

LNEPS = 1e-5
_NW = 32
_CH = 80
_P = 4


def _sc_gather(table, idx):
    n = idx.shape[0]
    d = table.shape[1]
    per_w = n // _NW
    n_ch = per_w // _CH
    assert per_w % _CH == 0 and n_ch % 2 == 0 and n % _NW == 0
    idx3 = idx.reshape(_NW, n_ch, _CH)
    mesh = plsc.VectorSubcoreMesh(core_axis_name="c", subcore_axis_name="s")

    @functools.partial(
        pl.kernel,
        mesh=mesh,
        out_type=jax.ShapeDtypeStruct((n, d), jnp.float32),
        scratch_types=[
            pltpu.VMEM((n_ch, _CH), jnp.int32),
            pltpu.VMEM((_CH, d), jnp.float32),
            pltpu.VMEM((_CH, d), jnp.float32),
            pltpu.SemaphoreType.DMA,
            pltpu.SemaphoreType.DMA,
            pltpu.SemaphoreType.DMA,
            pltpu.SemaphoreType.DMA,
        ],
    )
    def gather_kernel(table_hbm, idx_hbm, out_hbm, idx_v, buf0, buf1,
                      sg0, sg1, ss0, ss1):
        wid = lax.axis_index("s") * 2 + lax.axis_index("c")
        base = wid * per_w
        pltpu.sync_copy(idx_hbm.at[wid], idx_v)

        def body(g, _):
            j0 = 2 * g
            j1 = j0 + 1
            g0 = pltpu.make_async_copy(table_hbm.at[idx_v.at[j0]], buf0, sg0)
            g1 = pltpu.make_async_copy(table_hbm.at[idx_v.at[j1]], buf1, sg1)
            g0.start()
            g1.start()
            g0.wait()
            s0 = pltpu.make_async_copy(
                buf0, out_hbm.at[pl.ds(base + j0 * _CH, _CH)], ss0)
            s0.start()
            g1.wait()
            s1 = pltpu.make_async_copy(
                buf1, out_hbm.at[pl.ds(base + j1 * _CH, _CH)], ss1)
            s1.start()
            s0.wait()
            s1.wait()
            return 0

        lax.fori_loop(0, n_ch // 2, body, 0)

    return gather_kernel(table, idx3)


def _tc_fuse_chunk(xg, tids3, pos_table, type_table, gamma2, beta2, mask3,
                   b, s, d, bp, off, carry):
    g = 32
    ntypes = type_table.shape[0]
    ob = off // g

    def body(*refs):
        (x_ref, tid_ref, pos_ref, typ_ref, g_ref, b_ref, m_ref) = refs[:7]
        o_ref, mo_ref = refs[-2:]
        x = x_ref[...].reshape(g, s, d)
        emb = x + pos_ref[...].reshape(1, s, d)
        tid = tid_ref[...].astype(jnp.int32)
        for k in range(ntypes):
            row = typ_ref[k:k + 1, :].reshape(1, 1, d)
            emb = emb + jnp.where(tid == k, 1.0, 0.0) * row
        mean = jnp.mean(emb, axis=-1, keepdims=True)
        c = emb - mean
        var = jnp.mean(c * c, axis=-1, keepdims=True)
        y = c * lax.rsqrt(var + LNEPS)
        y = y * g_ref[...].reshape(1, 1, d) + b_ref[...].reshape(1, 1, d)
        o_ref[...] = y
        mo_ref[...] = ((1.0 - m_ref[...]) * -10000.0).reshape(g, 1, 1, s)

    in_specs = [
        pl.BlockSpec((g * s, d), lambda i: (i, 0)),
        pl.BlockSpec((g, s, 1), lambda i: (i + ob, 0, 0)),
        pl.BlockSpec((s, d), lambda i: (0, 0)),
        pl.BlockSpec((ntypes, d), lambda i: (0, 0)),
        pl.BlockSpec((1, d), lambda i: (0, 0)),
        pl.BlockSpec((1, d), lambda i: (0, 0)),
        pl.BlockSpec((g, 1, s), lambda i: (i + ob, 0, 0)),
    ]
    args = [xg, tids3, pos_table, type_table, gamma2, beta2, mask3]
    aliases = {}
    if carry is not None:
        in_specs += [pl.BlockSpec(memory_space=pl.ANY),
                     pl.BlockSpec(memory_space=pl.ANY)]
        args += [carry[0], carry[1]]
        aliases = {7: 0, 8: 1}

    return pl.pallas_call(
        body,
        grid=(bp // g,),
        in_specs=in_specs,
        out_specs=[
            pl.BlockSpec((g, s, d), lambda i: (i + ob, 0, 0)),
            pl.BlockSpec((g, 1, 1, s), lambda i: (i + ob, 0, 0, 0)),
        ],
        out_shape=[
            jax.ShapeDtypeStruct((b, s, d), jnp.float32),
            jax.ShapeDtypeStruct((b, 1, 1, s), jnp.float32),
        ],
        input_output_aliases=aliases,
    )(*args)


def kernel(input_ids, domain_type_ids, position_ids, attention_mask,
           token_table, pos_table, type_table, gamma, beta):
    b, s = input_ids.shape
    d = token_table.shape[1]
    bp = b // _P
    ids = input_ids.astype(jnp.int32)
    tids3 = domain_type_ids.astype(jnp.int8).reshape(b, s, 1)
    mask3 = attention_mask.astype(jnp.float32).reshape(b, 1, s)
    gamma2 = gamma.reshape(1, d)
    beta2 = beta.reshape(1, d)

    carry = None
    for p in range(_P):
        ids_p = lax.slice_in_dim(ids, p * bp, (p + 1) * bp, axis=0)
        g_p = _sc_gather(token_table, ids_p.reshape(-1))
        carry = _tc_fuse_chunk(g_p, tids3, pos_table, type_table,
                               gamma2, beta2, mask3, b, s, d,
                               bp, p * bp, carry)
    return (carry[0], carry[1])
